# initial kernel scaffold (unmeasured)
import jax
import jax.numpy as jnp
from jax import lax
from jax.experimental import pallas as pl
from jax.experimental.pallas import tpu as pltpu

N_DEV = 4
HEADS = 16
HALF = HEADS // 2
SCALE = 128 ** -0.5


def kernel(Q, K, V):
    _, s, h, d = Q.shape

    q = jnp.transpose(Q[0], (1, 0, 2))
    k = jnp.transpose(K[0], (1, 0, 2))
    v = jnp.transpose(V[0], (1, 0, 2))
    cw_in = jnp.stack([k[:HALF], v[:HALF]])
    ccw_in = jnp.stack([k[HALF:], v[HALF:]])

    def body(q_ref, cw_in_ref, ccw_in_ref, out_ref,
             cw_comm, ccw_comm, l_ref,
             cw_send, cw_recv, ccw_send, ccw_recv):
        my = lax.axis_index("i")
        right = lax.rem(my + 1, N_DEV)
        left = lax.rem(my + N_DEV - 1, N_DEV)

        barrier_sem = pltpu.get_barrier_semaphore()
        for nbr in (left, right):
            pl.semaphore_signal(
                barrier_sem, inc=1,
                device_id=(nbr,), device_id_type=pl.DeviceIdType.MESH,
            )
        pl.semaphore_wait(barrier_sem, 2)

        out_ref[...] = jnp.zeros_like(out_ref)
        l_ref[...] = jnp.zeros_like(l_ref)

        def accumulate(kv_ref, head_off):
            def head_body(hh, carry):
                hidx = head_off + hh
                qh = q_ref[hidx]
                kh = kv_ref[0, hh]
                vh = kv_ref[1, hh]
                sc = lax.dot_general(
                    qh, kh, (((1,), (1,)), ((), ())),
                    preferred_element_type=jnp.float32,
                )
                p = jnp.exp(sc * SCALE)
                l_ref[hidx, :] = l_ref[hidx, :] + jnp.sum(p, axis=1)
                out_ref[hidx] = out_ref[hidx] + lax.dot_general(
                    p, vh, (((1,), (0,)), ((), ())),
                    preferred_element_type=jnp.float32,
                )
                return carry
            lax.fori_loop(0, HALF, head_body, 0)

        def make_hop(hop):
            cw_src = cw_in_ref if hop == 1 else cw_comm.at[hop - 2]
            ccw_src = ccw_in_ref if hop == 1 else ccw_comm.at[hop - 2]
            cw = pltpu.make_async_remote_copy(
                src_ref=cw_src, dst_ref=cw_comm.at[hop - 1],
                send_sem=cw_send.at[hop - 1], recv_sem=cw_recv.at[hop - 1],
                device_id=(right,), device_id_type=pl.DeviceIdType.MESH,
            )
            ccw = pltpu.make_async_remote_copy(
                src_ref=ccw_src, dst_ref=ccw_comm.at[hop - 1],
                send_sem=ccw_send.at[hop - 1], recv_sem=ccw_recv.at[hop - 1],
                device_id=(left,), device_id_type=pl.DeviceIdType.MESH,
            )
            return cw, ccw

        for hop in range(1, N_DEV):
            cw, ccw = make_hop(hop)
            cw.start()
            ccw.start()
            if hop == 1:
                accumulate(cw_in_ref, 0)
                accumulate(ccw_in_ref, HALF)
            else:
                accumulate(cw_comm.at[hop - 2], 0)
                accumulate(ccw_comm.at[hop - 2], HALF)
            cw.wait()
            ccw.wait()

        accumulate(cw_comm.at[N_DEV - 2], 0)
        accumulate(ccw_comm.at[N_DEV - 2], HALF)

        def norm_body(hh, carry):
            out_ref[hh] = out_ref[hh] / l_ref[hh, :][:, None]
            return carry
        lax.fori_loop(0, HEADS, norm_body, 0)

    out = pl.pallas_call(
        body,
        out_shape=jax.ShapeDtypeStruct((h, s, d), jnp.float32),
        in_specs=[pl.BlockSpec(memory_space=pltpu.VMEM)] * 3,
        out_specs=pl.BlockSpec(memory_space=pltpu.VMEM),
        scratch_shapes=[
            pltpu.VMEM((N_DEV - 1, 2, HALF, s, d), jnp.float32),
            pltpu.VMEM((N_DEV - 1, 2, HALF, s, d), jnp.float32),
            pltpu.VMEM((HEADS, s), jnp.float32),
            pltpu.SemaphoreType.DMA((N_DEV - 1,)),
            pltpu.SemaphoreType.DMA((N_DEV - 1,)),
            pltpu.SemaphoreType.DMA((N_DEV - 1,)),
            pltpu.SemaphoreType.DMA((N_DEV - 1,)),
        ],
        compiler_params=pltpu.CompilerParams(collective_id=0),
    )(q, cw_in, ccw_in)

    return jnp.transpose(out, (1, 0, 2))[None]


# baseline (device time: 354666 ns/iter reference)
import jax
import jax.numpy as jnp
from jax import lax
from jax.experimental import pallas as pl
from jax.experimental.pallas import tpu as pltpu

N_DEV = 4
HEADS = 16
HALF = HEADS // 2
SCALE = 128 ** -0.5


def kernel(Q, K, V):
    _, s, h, d = Q.shape

    q = jnp.transpose(Q[0], (1, 0, 2))
    k = jnp.transpose(K[0], (1, 0, 2))
    v = jnp.transpose(V[0], (1, 0, 2))
    cw_in = jnp.stack([k[:HALF], v[:HALF]])
    ccw_in = jnp.stack([k[HALF:], v[HALF:]])

    def body(q_ref, cw_in_ref, ccw_in_ref, out_ref,
             cw_buf, ccw_buf, l_ref, local_sems,
             cw_send, cw_recv, ccw_send, ccw_recv,
             cw_credit, ccw_credit):
        my = lax.axis_index("i")
        right = lax.rem(my + 1, N_DEV)
        left = lax.rem(my + N_DEV - 1, N_DEV)

        lcw = pltpu.make_async_copy(cw_in_ref, cw_buf.at[0], local_sems.at[0])
        lccw = pltpu.make_async_copy(ccw_in_ref, ccw_buf.at[0], local_sems.at[1])
        lcw.start()
        lccw.start()

        barrier_sem = pltpu.get_barrier_semaphore()
        for nbr in (left, right):
            pl.semaphore_signal(
                barrier_sem, inc=1,
                device_id=(nbr,), device_id_type=pl.DeviceIdType.MESH,
            )
        pl.semaphore_wait(barrier_sem, 2)

        def rdma(src, dst, ssem, rsem, dev):
            return pltpu.make_async_remote_copy(
                src_ref=src, dst_ref=dst, send_sem=ssem, recv_sem=rsem,
                device_id=(dev,), device_id_type=pl.DeviceIdType.MESH,
            )

        cw1 = rdma(cw_in_ref, cw_buf.at[1], cw_send.at[0], cw_recv.at[0], right)
        ccw1 = rdma(ccw_in_ref, ccw_buf.at[1], ccw_send.at[0], ccw_recv.at[0], left)
        cw1.start()
        ccw1.start()

        out_ref[...] = jnp.zeros(out_ref.shape, out_ref.dtype)
        l_ref[...] = jnp.zeros(l_ref.shape, l_ref.dtype)

        def accumulate(kv_ref, head_off):
            def head_body(hh, carry):
                hidx = head_off + hh
                qh = q_ref[hidx]
                kh = kv_ref[0, hh]
                vh = kv_ref[1, hh]
                sc = lax.dot_general(
                    qh, kh, (((1,), (1,)), ((), ())),
                    preferred_element_type=jnp.float32,
                )
                p = jnp.exp(sc * SCALE)
                l_ref[hidx, :] = l_ref[hidx, :] + jnp.sum(p, axis=1)
                out_ref[hidx] = out_ref[hidx] + lax.dot_general(
                    p, vh, (((1,), (0,)), ((), ())),
                    preferred_element_type=jnp.float32,
                )
                return carry
            lax.fori_loop(0, HALF, head_body, 0)

        def credit_signal():
            pl.semaphore_signal(cw_credit, inc=1, device_id=(left,),
                                device_id_type=pl.DeviceIdType.MESH)
            pl.semaphore_signal(ccw_credit, inc=1, device_id=(right,),
                                device_id_type=pl.DeviceIdType.MESH)

        def credit_wait():
            pl.semaphore_wait(cw_credit, 1)
            pl.semaphore_wait(ccw_credit, 1)

        lcw.wait()
        lccw.wait()
        accumulate(cw_buf.at[0], 0)
        accumulate(ccw_buf.at[0], HALF)
        credit_signal()
        cw1.wait_recv()
        ccw1.wait_recv()

        credit_wait()
        cw2 = rdma(cw_buf.at[1], cw_buf.at[0], cw_send.at[1], cw_recv.at[1], right)
        ccw2 = rdma(ccw_buf.at[1], ccw_buf.at[0], ccw_send.at[1], ccw_recv.at[1], left)
        cw2.start()
        ccw2.start()
        accumulate(cw_buf.at[1], 0)
        accumulate(ccw_buf.at[1], HALF)
        cw1.wait_send()
        ccw1.wait_send()
        cw2.wait_send()
        ccw2.wait_send()
        credit_signal()
        cw2.wait_recv()
        ccw2.wait_recv()

        credit_wait()
        cw3 = rdma(cw_buf.at[0], cw_buf.at[1], cw_send.at[2], cw_recv.at[2], right)
        ccw3 = rdma(ccw_buf.at[0], ccw_buf.at[1], ccw_send.at[2], ccw_recv.at[2], left)
        cw3.start()
        ccw3.start()
        accumulate(cw_buf.at[0], 0)
        accumulate(ccw_buf.at[0], HALF)
        cw3.wait_recv()
        ccw3.wait_recv()
        accumulate(cw_buf.at[1], 0)
        accumulate(ccw_buf.at[1], HALF)
        cw3.wait_send()
        ccw3.wait_send()

        def norm_body(hh, carry):
            out_ref[hh] = out_ref[hh] / l_ref[hh, :][:, None]
            return carry
        lax.fori_loop(0, HEADS, norm_body, 0)

    out = pl.pallas_call(
        body,
        out_shape=jax.ShapeDtypeStruct((h, s, d), jnp.float32),
        in_specs=[
            pl.BlockSpec(memory_space=pltpu.VMEM),
            pl.BlockSpec(memory_space=pltpu.MemorySpace.HBM),
            pl.BlockSpec(memory_space=pltpu.MemorySpace.HBM),
        ],
        out_specs=pl.BlockSpec(memory_space=pltpu.VMEM),
        scratch_shapes=[
            pltpu.VMEM((2, 2, HALF, s, d), jnp.float32),
            pltpu.VMEM((2, 2, HALF, s, d), jnp.float32),
            pltpu.VMEM((HEADS, s), jnp.float32),
            pltpu.SemaphoreType.DMA((2,)),
            pltpu.SemaphoreType.DMA((3,)),
            pltpu.SemaphoreType.DMA((3,)),
            pltpu.SemaphoreType.DMA((3,)),
            pltpu.SemaphoreType.DMA((3,)),
            pltpu.SemaphoreType.REGULAR,
            pltpu.SemaphoreType.REGULAR,
        ],
        compiler_params=pltpu.CompilerParams(
            collective_id=0, vmem_limit_bytes=44 * 1024 * 1024,
        ),
    )(q, cw_in, ccw_in)

    return jnp.transpose(out, (1, 0, 2))[None]
